# baseline (speedup 1.0000x reference)
import functools

import jax
import jax.numpy as jnp
from jax import lax
from jax.experimental import pallas as pl
from jax.experimental.pallas import tpu as pltpu
from jax.experimental.pallas import tpu_sc as plsc

N, D, E, H = 10000, 128, 320000, 8
NC, NS, L = 2, 16, 16
NW = NC * NS
EPW = E // NW
NB = EPW // L
NR = N // 16

_HI_MASK = -65536


def _rne_bits(f):
    b = lax.bitcast_convert_type(f, jnp.int32)
    return b + 0x7FFF + jnp.bitwise_and(lax.shift_right_logical(b, 16), 1)


def _table_kernel(x_ref, wl_ref, wh_ref, bl_ref, bh_ref, out_ref):
    xv = x_ref[...]
    lo = jnp.dot(xv, wl_ref[...], preferred_element_type=jnp.float32) + bl_ref[...]
    hi = jnp.dot(xv, wh_ref[...], preferred_element_type=jnp.float32) + bh_ref[...]
    rl = _rne_bits(lo)
    rh = _rne_bits(hi)
    out_ref[...] = jnp.bitwise_or(
        jnp.bitwise_and(rh, _HI_MASK),
        jnp.bitwise_and(lax.shift_right_logical(rl, 16), 0xFFFF))


def _edge_body(tab_hbm, ei_hbm, par_hbm, out_hbm,
               tab_v, col_v, row_v, par_v, out_v):
    wid = lax.axis_index("s") * NC + lax.axis_index("c")
    base = wid * EPW
    pltpu.sync_copy(tab_hbm, tab_v)
    pltpu.sync_copy(ei_hbm.at[pl.ds(E + base, EPW)], col_v)
    pltpu.sync_copy(ei_hbm.at[pl.ds(base, EPW)], row_v)
    pltpu.sync_copy(par_hbm, par_v)

    w2 = [par_v[k, :] for k in range(H)]
    b2row = par_v[H, :]

    def tanh(s):
        e = jnp.exp(s + s)
        return 1.0 - 2.0 / (e + 1.0)

    @plsc.parallel_loop(0, EPW, step=L, unroll=8)
    def _loop(i):
        vc = col_v[pl.ds(i, L)]
        vr = row_v[pl.ds(i, L)]
        ca = vc * 8
        cb = vr * 8 + 4
        acc = b2row
        for j in range(4):
            wa = plsc.load_gather(tab_v, [ca + j])
            wb = plsc.load_gather(tab_v, [cb + j])
            a0 = plsc.bitcast(jnp.left_shift(wa, 16), jnp.float32)
            a1 = plsc.bitcast(jnp.bitwise_and(wa, _HI_MASK), jnp.float32)
            b0 = plsc.bitcast(jnp.left_shift(wb, 16), jnp.float32)
            b1v = plsc.bitcast(jnp.bitwise_and(wb, _HI_MASK), jnp.float32)
            t0 = tanh(a0 + b0)
            t1 = tanh(a1 + b1v)
            acc = acc + t0 * w2[j] + t1 * w2[j + 4]
        out_v[pl.ds(i, L)] = 1.0 / (1.0 + jnp.exp(-acc))
    pltpu.sync_copy(out_v, out_hbm.at[pl.ds(base, EPW)])


@functools.partial(
    pl.kernel,
    out_type=jax.ShapeDtypeStruct((E,), jnp.float32),
    mesh=plsc.VectorSubcoreMesh(core_axis_name="c", subcore_axis_name="s",
                                num_cores=NC, num_subcores=NS),
    scratch_types=[
        pltpu.VMEM((N * H,), jnp.int32),
        pltpu.VMEM((EPW,), jnp.int32),
        pltpu.VMEM((EPW,), jnp.int32),
        pltpu.VMEM((H + 1, L), jnp.float32),
        pltpu.VMEM((EPW,), jnp.float32),
    ],
    compiler_params=pltpu.CompilerParams(needs_layout_passes=False),
)
def _edge_mlp(tab_hbm, ei_hbm, par_hbm, out_hbm,
              tab_v, col_v, row_v, par_v, out_v):
    _edge_body(tab_hbm, ei_hbm, par_hbm, out_hbm,
               tab_v, col_v, row_v, par_v, out_v)


def kernel(x, edge_index, W1, b1, W2, b2):
    WL = jnp.concatenate([W1[:D, 0:4], W1[D:, 0:4]], axis=1)
    WH = jnp.concatenate([W1[:D, 4:8], W1[D:, 4:8]], axis=1)
    eye16 = jnp.eye(16, dtype=jnp.float32)
    WLb = jnp.kron(eye16, WL)
    WHb = jnp.kron(eye16, WH)
    z4 = jnp.zeros((4,), jnp.float32)
    bL = jnp.tile(jnp.concatenate([b1[0:4], z4]), 16).reshape(1, 128)
    bH = jnp.tile(jnp.concatenate([b1[4:8], z4]), 16).reshape(1, 128)

    Tp = pl.pallas_call(
        _table_kernel,
        out_shape=jax.ShapeDtypeStruct((NR, 128), jnp.int32),
    )(x.reshape(NR, 16 * D), WLb, WHb, bL, bH)

    par = jnp.concatenate(
        [jnp.broadcast_to(W2.reshape(H, 1), (H, L)),
         jnp.broadcast_to(b2.reshape(1, 1), (1, L))], axis=0)

    out = _edge_mlp(Tp.reshape(N * H), edge_index.reshape(2 * E), par)
    return out.reshape(E, 1)


# speedup vs baseline: 1.6233x; 1.6233x over previous
import functools

import jax
import jax.numpy as jnp
from jax import lax
from jax.experimental import pallas as pl
from jax.experimental.pallas import tpu as pltpu
from jax.experimental.pallas import tpu_sc as plsc

N, D, E, H = 10000, 128, 320000, 8
NC, NS, L = 2, 16, 16
NW = NC * NS
EPW = E // NW
NB = EPW // L
NR = N // 16

_HI_MASK = -65536


def _rne_bits(f):
    b = lax.bitcast_convert_type(f, jnp.int32)
    return b + 0x7FFF + jnp.bitwise_and(lax.shift_right_logical(b, 16), 1)


def _table_kernel(x_ref, wl_ref, wh_ref, bl_ref, bh_ref, out_ref):
    xv = x_ref[...]
    lo = jnp.dot(xv, wl_ref[...], preferred_element_type=jnp.float32) + bl_ref[...]
    hi = jnp.dot(xv, wh_ref[...], preferred_element_type=jnp.float32) + bh_ref[...]
    rl = _rne_bits(lo)
    rh = _rne_bits(hi)
    out_ref[...] = jnp.bitwise_or(
        jnp.bitwise_and(rh, _HI_MASK),
        jnp.bitwise_and(lax.shift_right_logical(rl, 16), 0xFFFF))


def _edge_body(tab_hbm, ei_hbm, par_hbm, out_hbm,
               tab_v, col_v, row_v, par_v, out_v):
    wid = lax.axis_index("s") * NC + lax.axis_index("c")
    base = wid * EPW
    pltpu.sync_copy(tab_hbm, tab_v)
    pltpu.sync_copy(ei_hbm.at[pl.ds(E + base, EPW)], col_v)
    pltpu.sync_copy(ei_hbm.at[pl.ds(base, EPW)], row_v)
    pltpu.sync_copy(par_hbm, par_v)

    w2 = [par_v[k, :] for k in range(H)]
    b2row = par_v[H, :]

    def tanh(s):
        e = jnp.exp(s + s)
        return 1.0 - 2.0 / (e + 1.0)

    @plsc.parallel_loop(0, EPW, step=L, unroll=4)
    def _loop(i):
        vc = col_v[pl.ds(i, L)]
        vr = row_v[pl.ds(i, L)]
        ca = vc * 8
        cb = vr * 8 + 4
        acc = b2row
        for j in range(4):
            wa = plsc.load_gather(tab_v, [ca + j])
            wb = plsc.load_gather(tab_v, [cb + j])
            a0 = plsc.bitcast(jnp.left_shift(wa, 16), jnp.float32)
            a1 = plsc.bitcast(jnp.bitwise_and(wa, _HI_MASK), jnp.float32)
            b0 = plsc.bitcast(jnp.left_shift(wb, 16), jnp.float32)
            b1v = plsc.bitcast(jnp.bitwise_and(wb, _HI_MASK), jnp.float32)
            t0 = tanh(a0 + b0)
            t1 = tanh(a1 + b1v)
            acc = acc + t0 * w2[j] + t1 * w2[j + 4]
        out_v[pl.ds(i, L)] = 1.0 / (1.0 + jnp.exp(-acc))
    pltpu.sync_copy(out_v, out_hbm.at[pl.ds(base, EPW)])


@functools.partial(
    pl.kernel,
    out_type=jax.ShapeDtypeStruct((E,), jnp.float32),
    mesh=plsc.VectorSubcoreMesh(core_axis_name="c", subcore_axis_name="s",
                                num_cores=NC, num_subcores=NS),
    scratch_types=[
        pltpu.VMEM((N * H,), jnp.int32),
        pltpu.VMEM((EPW,), jnp.int32),
        pltpu.VMEM((EPW,), jnp.int32),
        pltpu.VMEM((H + 1, L), jnp.float32),
        pltpu.VMEM((EPW,), jnp.float32),
    ],
    compiler_params=pltpu.CompilerParams(needs_layout_passes=False),
)
def _edge_mlp(tab_hbm, ei_hbm, par_hbm, out_hbm,
              tab_v, col_v, row_v, par_v, out_v):
    _edge_body(tab_hbm, ei_hbm, par_hbm, out_hbm,
               tab_v, col_v, row_v, par_v, out_v)


def kernel(x, edge_index, W1, b1, W2, b2):
    WL = jnp.concatenate([W1[:D, 0:4], W1[D:, 0:4]], axis=1)
    WH = jnp.concatenate([W1[:D, 4:8], W1[D:, 4:8]], axis=1)
    z4 = jnp.zeros((4,), jnp.float32)
    bL = jnp.concatenate([b1[0:4], z4]).reshape(1, H)
    bH = jnp.concatenate([b1[4:8], z4]).reshape(1, H)

    Tp = pl.pallas_call(
        _table_kernel,
        out_shape=jax.ShapeDtypeStruct((N, H), jnp.int32),
    )(x, WL, WH, bL, bH)

    par = jnp.concatenate(
        [jnp.broadcast_to(W2.reshape(H, 1), (H, L)),
         jnp.broadcast_to(b2.reshape(1, 1), (1, L))], axis=0)

    out = _edge_mlp(Tp.reshape(N * H), edge_index.reshape(2 * E), par)
    return out.reshape(E, 1)


# speedup vs baseline: 1.6421x; 1.0116x over previous
import functools

import jax
import jax.numpy as jnp
from jax import lax
from jax.experimental import pallas as pl
from jax.experimental.pallas import tpu as pltpu
from jax.experimental.pallas import tpu_sc as plsc

N, D, E, H = 10000, 128, 320000, 8
NC, NS, L = 2, 16, 16
NW = NC * NS
EPW = E // NW
NB = EPW // L
NR = N // 16

_HI_MASK = -65536


def _rne_bits(f):
    b = lax.bitcast_convert_type(f, jnp.int32)
    return b + 0x7FFF + jnp.bitwise_and(lax.shift_right_logical(b, 16), 1)


def _table_kernel(x_ref, wl_ref, wh_ref, bl_ref, bh_ref, out_ref):
    xv = x_ref[...]
    lo = jnp.dot(xv, wl_ref[...], preferred_element_type=jnp.float32) + bl_ref[...]
    hi = jnp.dot(xv, wh_ref[...], preferred_element_type=jnp.float32) + bh_ref[...]
    rl = _rne_bits(lo)
    rh = _rne_bits(hi)
    out_ref[...] = jnp.bitwise_or(
        jnp.bitwise_and(rh, _HI_MASK),
        jnp.bitwise_and(lax.shift_right_logical(rl, 16), 0xFFFF))


def _edge_body(tab_hbm, ei_hbm, par_hbm, out_hbm,
               tab_v, col_v, row_v, par_v, out_v):
    wid = lax.axis_index("s") * NC + lax.axis_index("c")
    base = wid * EPW
    pltpu.sync_copy(tab_hbm, tab_v)
    pltpu.sync_copy(ei_hbm.at[pl.ds(E + base, EPW)], col_v)
    pltpu.sync_copy(ei_hbm.at[pl.ds(base, EPW)], row_v)
    pltpu.sync_copy(par_hbm, par_v)

    w2 = [par_v[k, :] for k in range(H)]
    acc0 = par_v[H, :]

    @plsc.parallel_loop(0, EPW, step=L, unroll=4)
    def _loop(i):
        vc = col_v[pl.ds(i, L)]
        vr = row_v[pl.ds(i, L)]
        cb = vr + 4
        acc = acc0
        for j in range(4):
            wa = plsc.load_gather(tab_v, [vc + j if j else vc])
            wb = plsc.load_gather(tab_v, [cb + j if j else cb])
            a0 = plsc.bitcast(jnp.left_shift(wa, 16), jnp.float32)
            a1 = plsc.bitcast(jnp.bitwise_and(wa, _HI_MASK), jnp.float32)
            b0 = plsc.bitcast(jnp.left_shift(wb, 16), jnp.float32)
            b1v = plsc.bitcast(jnp.bitwise_and(wb, _HI_MASK), jnp.float32)
            r0 = 1.0 / (jnp.exp(a0 + b0) + 1.0)
            r1 = 1.0 / (jnp.exp(a1 + b1v) + 1.0)
            acc = acc + r0 * w2[j] + r1 * w2[j + 4]
        out_v[pl.ds(i, L)] = 1.0 / (1.0 + jnp.exp(acc))
    pltpu.sync_copy(out_v, out_hbm.at[pl.ds(base, EPW)])


@functools.partial(
    pl.kernel,
    out_type=jax.ShapeDtypeStruct((E,), jnp.float32),
    mesh=plsc.VectorSubcoreMesh(core_axis_name="c", subcore_axis_name="s",
                                num_cores=NC, num_subcores=NS),
    scratch_types=[
        pltpu.VMEM((N * H,), jnp.int32),
        pltpu.VMEM((EPW,), jnp.int32),
        pltpu.VMEM((EPW,), jnp.int32),
        pltpu.VMEM((H + 1, L), jnp.float32),
        pltpu.VMEM((EPW,), jnp.float32),
    ],
    compiler_params=pltpu.CompilerParams(needs_layout_passes=False),
)
def _edge_mlp(tab_hbm, ei_hbm, par_hbm, out_hbm,
              tab_v, col_v, row_v, par_v, out_v):
    _edge_body(tab_hbm, ei_hbm, par_hbm, out_hbm,
               tab_v, col_v, row_v, par_v, out_v)


_C2 = 2.0
_NL2E = -1.0


def kernel(x, edge_index, W1, b1, W2, b2):
    WL = _C2 * jnp.concatenate([W1[:D, 0:4], W1[D:, 0:4]], axis=1)
    WH = _C2 * jnp.concatenate([W1[:D, 4:8], W1[D:, 4:8]], axis=1)
    z4 = jnp.zeros((4,), jnp.float32)
    bL = (_C2 * jnp.concatenate([b1[0:4], z4])).reshape(1, H)
    bH = (_C2 * jnp.concatenate([b1[4:8], z4])).reshape(1, H)

    Tp = pl.pallas_call(
        _table_kernel,
        out_shape=jax.ShapeDtypeStruct((N, H), jnp.int32),
    )(x, WL, WH, bL, bH)

    acc0 = _NL2E * (jnp.sum(W2) + b2[0])
    par = jnp.concatenate(
        [jnp.broadcast_to(_C2 * W2.reshape(H, 1), (H, L)),
         jnp.broadcast_to(acc0.reshape(1, 1), (1, L))], axis=0)

    out = _edge_mlp(Tp.reshape(N * H), edge_index.reshape(2 * E) * 8, par)
    return out.reshape(E, 1)
